# Initial kernel scaffold; baseline (speedup 1.0000x reference)
import functools

import jax
import jax.numpy as jnp
from jax import lax
from jax.experimental import pallas as pl
from jax.experimental.pallas import tpu as pltpu
from jax.experimental.pallas import tpu_sc as plsc

N_NODES_K = 10000
N_EDGES_K = 320000
NC = 2
NS = 16
NW = NC * NS
EPW = N_EDGES_K // NW
CH = 125
NCHUNK = EPW // CH
ROWS_PER_TILE = N_NODES_K // NS
DEG_W = 16

_mesh = plsc.VectorSubcoreMesh(core_axis_name="c", subcore_axis_name="s")


def _zero_rows(ref, nrows, width):
    def body(i, _):
        for k in range(width // 16):
            ref[i, pl.ds(k * 16, 16)] = jnp.zeros((16,), jnp.float32)
        return 0

    lax.fori_loop(0, nrows, body, 0)


@functools.partial(
    pl.kernel,
    out_type=jax.ShapeDtypeStruct((NC, N_NODES_K, DEG_W), jnp.float32),
    mesh=_mesh,
    scratch_types=[
        pltpu.VMEM((NCHUNK, CH), jnp.int32),
        pltpu.VMEM((CH, DEG_W), jnp.float32),
        pltpu.VMEM((CH, DEG_W), jnp.float32),
        pltpu.VMEM_SHARED((N_NODES_K, DEG_W), jnp.float32),
    ],
)
def _deg_partials(dst3_hbm, out_hbm, dst_v, ones_v, zbuf_v, acc_sh):
    cid = lax.axis_index("c")
    tid = lax.axis_index("s")
    wid = tid * NC + cid

    def fill(i, _):
        ones_v[i, pl.ds(0, 16)] = jnp.full((16,), 1.0, jnp.float32)
        zbuf_v[i, pl.ds(0, 16)] = jnp.zeros((16,), jnp.float32)
        return 0

    lax.fori_loop(0, CH, fill, 0)

    row0 = tid * ROWS_PER_TILE
    for k in range(ROWS_PER_TILE // CH):
        pltpu.sync_copy(zbuf_v, acc_sh.at[pl.ds(row0 + k * CH, CH)])

    pltpu.sync_copy(dst3_hbm.at[wid], dst_v)

    plsc.subcore_barrier()

    def chunk(j, _):
        pltpu.sync_copy(ones_v, acc_sh.at[dst_v.at[j]], add=True)
        return 0

    lax.fori_loop(0, NCHUNK, chunk, 0)

    plsc.subcore_barrier()

    for k in range(ROWS_PER_TILE // CH):
        r = row0 + k * CH
        pltpu.sync_copy(acc_sh.at[pl.ds(r, CH)], zbuf_v)
        pltpu.sync_copy(zbuf_v, out_hbm.at[cid, pl.ds(r, CH)])


@functools.partial(
    pl.kernel,
    out_type=jax.ShapeDtypeStruct((NC, N_NODES_K, 64), jnp.float32),
    mesh=_mesh,
    scratch_types=[
        pltpu.VMEM((NCHUNK, CH), jnp.int32),
        pltpu.VMEM((NCHUNK, CH), jnp.int32),
        pltpu.VMEM((2, CH, 64), jnp.float32),
        pltpu.VMEM((CH, 64), jnp.float32),
        pltpu.VMEM_SHARED((N_NODES_K, 64), jnp.float32),
        pltpu.SemaphoreType.DMA,
    ],
)
def _msg_partials(ht_hbm, src3_hbm, dst3_hbm, out_hbm,
                  src_v, dst_v, rows_v, zbuf_v, acc_sh, sem):
    cid = lax.axis_index("c")
    tid = lax.axis_index("s")
    wid = tid * NC + cid

    _zero_rows(zbuf_v, CH, 64)

    row0 = tid * ROWS_PER_TILE
    for k in range(ROWS_PER_TILE // CH):
        pltpu.sync_copy(zbuf_v, acc_sh.at[pl.ds(row0 + k * CH, CH)])

    pltpu.sync_copy(src3_hbm.at[wid], src_v)
    pltpu.sync_copy(dst3_hbm.at[wid], dst_v)

    plsc.subcore_barrier()

    pltpu.async_copy(ht_hbm.at[src_v.at[0]], rows_v.at[0], sem)
    pltpu.async_copy(ht_hbm.at[src_v.at[1]], rows_v.at[1], sem)

    def group(g, _):
        j0 = 2 * g
        for b in range(2):
            j = j0 + b
            pltpu.make_async_copy(
                ht_hbm.at[src_v.at[0]], rows_v.at[b], sem).wait()
            pltpu.sync_copy(rows_v.at[b], acc_sh.at[dst_v.at[j]], add=True)

            @pl.when(j + 2 < NCHUNK)
            def _():
                pltpu.async_copy(ht_hbm.at[src_v.at[j + 2]], rows_v.at[b],
                                 sem)
        return 0

    lax.fori_loop(0, NCHUNK // 2, group, 0)

    plsc.subcore_barrier()

    for k in range(ROWS_PER_TILE // CH):
        r = row0 + k * CH
        pltpu.sync_copy(acc_sh.at[pl.ds(r, CH)], zbuf_v)
        pltpu.sync_copy(zbuf_v, out_hbm.at[cid, pl.ds(r, CH)])


def _tc1_body(degp_ref, x_ref, w1_ref, ht_ref, dinv_ref):
    degp = degp_ref[...]
    deg = degp[0, :, 0] + degp[1, :, 0] + 1.0
    dinv = lax.rsqrt(deg)
    h = jnp.dot(x_ref[...], w1_ref[...], preferred_element_type=jnp.float32)
    ht_ref[...] = h * dinv[:, None]
    dinv_ref[...] = dinv[:, None]


def _tc2_body(r_ref, ht_ref, dinv_ref, b1_ref, w2_ref, ht2_ref):
    r = r_ref[...]
    dinv = dinv_ref[...]
    out1 = dinv * (r[0] + r[1] + ht_ref[...]) + b1_ref[...]
    h1 = jnp.maximum(out1, 0.0)
    h2 = jnp.dot(h1, w2_ref[...], preferred_element_type=jnp.float32)
    ht2_ref[...] = h2 * dinv


def _tc3_body(r_ref, ht2_ref, dinv_ref, b2_ref, out_ref):
    r = r_ref[...]
    out_ref[...] = dinv_ref[...] * (r[0] + r[1] + ht2_ref[...]) + b2_ref[...]


@jax.jit
def kernel(x, edge_index, W1, b1, W2, b2):
    n = N_NODES_K
    src3 = edge_index[0].reshape(NW, NCHUNK, CH)
    dst3 = edge_index[1].reshape(NW, NCHUNK, CH)

    degp = _deg_partials(dst3)

    ht, dinv = pl.pallas_call(
        _tc1_body,
        out_shape=[
            jax.ShapeDtypeStruct((n, 64), jnp.float32),
            jax.ShapeDtypeStruct((n, 1), jnp.float32),
        ],
    )(degp, x, W1)

    r1 = _msg_partials(ht, src3, dst3)

    ht2 = pl.pallas_call(
        _tc2_body,
        out_shape=jax.ShapeDtypeStruct((n, 64), jnp.float32),
    )(r1, ht, dinv, b1.reshape(1, 64), W2)

    r2 = _msg_partials(ht2, src3, dst3)

    out = pl.pallas_call(
        _tc3_body,
        out_shape=jax.ShapeDtypeStruct((n, 64), jnp.float32),
    )(r2, ht2, dinv, b2.reshape(1, 64))

    return out


# speedup vs baseline: 39.0904x; 39.0904x over previous
import functools

import jax
import jax.numpy as jnp
from jax import lax
from jax.experimental import pallas as pl
from jax.experimental.pallas import tpu as pltpu
from jax.experimental.pallas import tpu_sc as plsc

N_NODES_K = 10000
N_EDGES_K = 320000
NC = 2
NS = 16
NW = NC * NS
EPW = N_EDGES_K // NW
CH = 125
NCHUNK = EPW // CH
STRIPE = 640
SUB = 80
DEG_W = 16

_mesh = plsc.VectorSubcoreMesh(core_axis_name="c", subcore_axis_name="s")
_sc_params = pltpu.CompilerParams(use_tc_tiling_on_sc=False)


def _zero_rows(ref, nrows, width):
    def body(i, _):
        for k in range(width // 16):
            ref[i, pl.ds(k * 16, 16)] = jnp.zeros((16,), jnp.float32)
        return 0

    lax.fori_loop(0, nrows, body, 0)


@functools.partial(
    pl.kernel,
    out_type=jax.ShapeDtypeStruct((NC, N_NODES_K, DEG_W), jnp.float32),
    mesh=_mesh,
    scratch_types=[
        pltpu.VMEM((NCHUNK, CH), jnp.int32),
        pltpu.VMEM((CH, DEG_W), jnp.float32),
        pltpu.VMEM((SUB, DEG_W), jnp.float32),
        pltpu.VMEM_SHARED((N_NODES_K, DEG_W), jnp.float32),
    ],
    compiler_params=_sc_params,
)
def _deg_partials(dst3_hbm, out_hbm, dst_v, ones_v, zbuf_v, acc_sh):
    cid = lax.axis_index("c")
    tid = lax.axis_index("s")
    wid = tid * NC + cid

    def fill(i, _):
        ones_v[i, pl.ds(0, 16)] = jnp.full((16,), 1.0, jnp.float32)
        return 0

    lax.fori_loop(0, CH, fill, 0)
    _zero_rows(zbuf_v, SUB, DEG_W)

    row0 = tid * STRIPE
    nsub = jnp.where(tid == NS - 1,
                     (N_NODES_K - (NS - 1) * STRIPE) // SUB, STRIPE // SUB)

    def zero_sub(k, _):
        r = pl.multiple_of(row0 + k * SUB, 8)
        pltpu.sync_copy(zbuf_v, acc_sh.at[pl.ds(r, SUB)])
        return 0

    lax.fori_loop(0, nsub, zero_sub, 0)

    pltpu.sync_copy(dst3_hbm.at[wid], dst_v)

    plsc.subcore_barrier()

    def chunk(j, _):
        pltpu.sync_copy(ones_v, acc_sh.at[dst_v.at[j]], add=True)
        return 0

    lax.fori_loop(0, NCHUNK, chunk, 0)

    plsc.subcore_barrier()

    def readback(k, _):
        r = pl.multiple_of(row0 + k * SUB, 8)
        pltpu.sync_copy(acc_sh.at[pl.ds(r, SUB)], zbuf_v)
        pltpu.sync_copy(zbuf_v, out_hbm.at[cid, pl.ds(r, SUB)])
        return 0

    lax.fori_loop(0, nsub, readback, 0)


@functools.partial(
    pl.kernel,
    out_type=jax.ShapeDtypeStruct((NC, N_NODES_K, 64), jnp.float32),
    mesh=_mesh,
    scratch_types=[
        pltpu.VMEM((NCHUNK, CH), jnp.int32),
        pltpu.VMEM((NCHUNK, CH), jnp.int32),
        pltpu.VMEM((2, CH, 64), jnp.float32),
        pltpu.VMEM((SUB, 64), jnp.float32),
        pltpu.VMEM_SHARED((N_NODES_K, 64), jnp.float32),
        pltpu.SemaphoreType.DMA,
    ],
    compiler_params=_sc_params,
)
def _msg_partials(ht_hbm, src3_hbm, dst3_hbm, out_hbm,
                  src_v, dst_v, rows_v, zbuf_v, acc_sh, sem):
    cid = lax.axis_index("c")
    tid = lax.axis_index("s")
    wid = tid * NC + cid

    _zero_rows(zbuf_v, SUB, 64)

    row0 = tid * STRIPE
    nsub = jnp.where(tid == NS - 1,
                     (N_NODES_K - (NS - 1) * STRIPE) // SUB, STRIPE // SUB)

    def zero_sub(k, _):
        r = pl.multiple_of(row0 + k * SUB, 8)
        pltpu.sync_copy(zbuf_v, acc_sh.at[pl.ds(r, SUB)])
        return 0

    lax.fori_loop(0, nsub, zero_sub, 0)

    pltpu.sync_copy(src3_hbm.at[wid], src_v)
    pltpu.sync_copy(dst3_hbm.at[wid], dst_v)

    plsc.subcore_barrier()

    pltpu.async_copy(ht_hbm.at[src_v.at[0]], rows_v.at[0], sem)
    pltpu.async_copy(ht_hbm.at[src_v.at[1]], rows_v.at[1], sem)

    def group(g, _):
        j0 = 2 * g
        for b in range(2):
            j = j0 + b
            pltpu.make_async_copy(
                ht_hbm.at[src_v.at[0]], rows_v.at[b], sem).wait()
            pltpu.sync_copy(rows_v.at[b], acc_sh.at[dst_v.at[j]], add=True)

            @pl.when(j + 2 < NCHUNK)
            def _():
                pltpu.async_copy(ht_hbm.at[src_v.at[j + 2]], rows_v.at[b],
                                 sem)
        return 0

    lax.fori_loop(0, NCHUNK // 2, group, 0)

    plsc.subcore_barrier()

    def readback(k, _):
        r = pl.multiple_of(row0 + k * SUB, 8)
        pltpu.sync_copy(acc_sh.at[pl.ds(r, SUB)], zbuf_v)
        pltpu.sync_copy(zbuf_v, out_hbm.at[cid, pl.ds(r, SUB)])
        return 0

    lax.fori_loop(0, nsub, readback, 0)


def _tc1_body(degp_ref, x_ref, w1_ref, ht_ref, dinv_ref):
    degp = degp_ref[...]
    deg = degp[0, :, 0] + degp[1, :, 0] + 1.0
    dinv = lax.rsqrt(deg)
    h = jnp.dot(x_ref[...], w1_ref[...], preferred_element_type=jnp.float32)
    ht_ref[...] = h * dinv[:, None]
    dinv_ref[...] = dinv[:, None]


def _tc2_body(r_ref, ht_ref, dinv_ref, b1_ref, w2_ref, ht2_ref):
    r = r_ref[...]
    dinv = dinv_ref[...]
    out1 = dinv * (r[0] + r[1] + ht_ref[...]) + b1_ref[...]
    h1 = jnp.maximum(out1, 0.0)
    h2 = jnp.dot(h1, w2_ref[...], preferred_element_type=jnp.float32)
    ht2_ref[...] = h2 * dinv


def _tc3_body(r_ref, ht2_ref, dinv_ref, b2_ref, out_ref):
    r = r_ref[...]
    out_ref[...] = dinv_ref[...] * (r[0] + r[1] + ht2_ref[...]) + b2_ref[...]


@jax.jit
def kernel(x, edge_index, W1, b1, W2, b2):
    n = N_NODES_K
    src3 = edge_index[0].reshape(NW, NCHUNK, CH)
    dst3 = edge_index[1].reshape(NW, NCHUNK, CH)

    degp = _deg_partials(dst3)

    ht, dinv = pl.pallas_call(
        _tc1_body,
        out_shape=[
            jax.ShapeDtypeStruct((n, 64), jnp.float32),
            jax.ShapeDtypeStruct((n, 1), jnp.float32),
        ],
    )(degp, x, W1)

    r1 = _msg_partials(ht, src3, dst3)

    ht2 = pl.pallas_call(
        _tc2_body,
        out_shape=jax.ShapeDtypeStruct((n, 64), jnp.float32),
    )(r1, ht, dinv, b1.reshape(1, 64), W2)

    r2 = _msg_partials(ht2, src3, dst3)

    out = pl.pallas_call(
        _tc3_body,
        out_shape=jax.ShapeDtypeStruct((n, 64), jnp.float32),
    )(r2, ht2, dinv, b2.reshape(1, 64))

    return out
